# baseline (speedup 1.0000x reference)
import jax
import jax.numpy as jnp
from jax.experimental import pallas as pl
from jax.experimental.pallas import tpu as pltpu

N = 2048
H = N // 2
BLK = 256
NBLK = N // BLK
F32 = jnp.float32


def _bn_relu(h, g, be):
    m = jnp.mean(h, axis=0, keepdims=True)
    v = jnp.mean((h - m) ** 2, axis=0, keepdims=True)
    return jax.nn.relu((h - m) / jnp.sqrt(v + 1e-5) * g + be)


def _enc_tail(hpre_ref, g1, be1, w2, b2, g2, be2, w3, b3, g3, be3, hs_ref):
    h = _bn_relu(hpre_ref[...].reshape(N, 256), g1[...], be1[...])
    h = jax.lax.dot_general(h, w2[...], (((1,), (1,)), ((), ())),
                            preferred_element_type=F32) + b2[...]
    h = _bn_relu(h, g2[...], be2[...])
    h = jax.lax.dot_general(h, w3[...], (((1,), (1,)), ((), ())),
                            preferred_element_type=F32) + b3[...]
    hs_ref[...] = _bn_relu(h, g3[...], be3[...])


def _mono_kernel(x1l_ref, x1r_ref, x2l_ref, x2r_ref,
                 a1l_ref, a1r_ref, a2l_ref, a2r_ref, all_ref, alr_ref,
                 w1al, w1ar, b1a, g1a, be1a, w2a, b2a, g2a, be2a,
                 w3a, b3a, g3a, be3a,
                 w1bl, w1br, b1b, g1b, be1b, w2b, b2b, g2b, be2b,
                 w3b, b3b, g3b, be3b,
                 w_ref, wc_ref, bc_ref,
                 out_ref, hpre1, hpre2, h1s, h2s, acc_ref):
    t = pl.program_id(0)

    @pl.when(t < NBLK)
    def _enc_step():
        dn = (((1,), (1,)), ((), ()))
        h1 = (jax.lax.dot_general(x1l_ref[...], w1al[...], dn,
                                  preferred_element_type=F32,
                                  precision=jax.lax.Precision.DEFAULT) +
              jax.lax.dot_general(x1r_ref[...], w1ar[...], dn,
                                  preferred_element_type=F32,
                                  precision=jax.lax.Precision.DEFAULT))
        hpre1[t] = h1 + b1a[...]
        h2 = (jax.lax.dot_general(x2l_ref[...], w1bl[...], dn,
                                  preferred_element_type=F32,
                                  precision=jax.lax.Precision.DEFAULT) +
              jax.lax.dot_general(x2r_ref[...], w1br[...], dn,
                                  preferred_element_type=F32,
                                  precision=jax.lax.Precision.DEFAULT))
        hpre2[t] = h2 + b1b[...]

    @pl.when(t == NBLK - 1)
    def _enc_tails():
        _enc_tail(hpre1, g1a, be1a, w2a, b2a, g2a, be2a,
                  w3a, b3a, g3a, be3a, h1s)
        _enc_tail(hpre2, g1b, be1b, w2b, b2b, g2b, be2b,
                  w3b, b3b, g3b, be3b, h2s)

    @pl.when(t == NBLK)
    def _init_acc():
        acc_ref[...] = jnp.zeros_like(acc_ref)

    @pl.when(t >= NBLK)
    def _attn_step():
        j = t - NBLK
        w = w_ref[...]
        dn = (((1,), (0,)), ((), ()))

        def attend(adjl, adjr, hs_ref):
            deg = (jnp.sum(adjl, axis=1, keepdims=True) +
                   jnp.sum(adjr, axis=1, keepdims=True))
            agg = (jax.lax.dot_general(all_ref[...] * adjl,
                                       hs_ref[:H, :], dn,
                                       preferred_element_type=F32,
                                       precision=jax.lax.Precision.DEFAULT) +
                   jax.lax.dot_general(alr_ref[...] * adjr,
                                       hs_ref[H:, :], dn,
                                       preferred_element_type=F32,
                                       precision=jax.lax.Precision.DEFAULT))
            inv = w / deg
            return agg * inv + hs_ref[pl.ds(j * BLK, BLK), :]

        new1 = attend(a1l_ref[...], a1r_ref[...], h1s)
        new2 = attend(a2l_ref[...], a2r_ref[...], h2s)
        contrib = (jnp.sum(wc_ref[:, 0] * new1[None], axis=(1, 2)) +
                   jnp.sum(wc_ref[:, 1] * new2[None], axis=(1, 2)))
        acc_ref[...] += contrib.reshape(1, 2)

        @pl.when(t == 2 * NBLK - 1)
        def _fin():
            logits = acc_ref[...] + bc_ref[...]
            e = jnp.exp(logits - jnp.max(logits))
            out_ref[...] = e / jnp.sum(e)


def kernel(x1, x2, adj1, adj2,
           enc1_W1, enc1_b1, enc1_g1, enc1_be1,
           enc1_W2, enc1_b2, enc1_g2, enc1_be2,
           enc1_W3, enc1_b3, enc1_g3, enc1_be3,
           enc2_W1, enc2_b1, enc2_g1, enc2_be1,
           enc2_W2, enc2_b2, enc2_g2, enc2_be2,
           enc2_W3, enc2_b3, enc2_g3, enc2_be3,
           W, alpha1, alpha2, Wc, bc):
    row = lambda a: a.reshape(1, -1)
    full = lambda a: pl.BlockSpec(a.shape, lambda t: (0,) * a.ndim)
    enc_l = pl.BlockSpec((BLK, H), lambda t: (jnp.clip(t, 0, NBLK - 1), 0))
    enc_r = pl.BlockSpec((BLK, H), lambda t: (jnp.clip(t, 0, NBLK - 1), 1))
    att_l = pl.BlockSpec((BLK, H),
                         lambda t: (jnp.clip(t - NBLK, 0, NBLK - 1), 0))
    att_r = pl.BlockSpec((BLK, H),
                         lambda t: (jnp.clip(t - NBLK, 0, NBLK - 1), 1))
    Wc4 = Wc.reshape(2, 2, N, 64)
    bc2 = bc.reshape(1, 2)
    smalls_a = (enc1_W1[:, :H], enc1_W1[:, H:],
                row(enc1_b1), row(enc1_g1), row(enc1_be1),
                enc1_W2, row(enc1_b2), row(enc1_g2), row(enc1_be2),
                enc1_W3, row(enc1_b3), row(enc1_g3), row(enc1_be3))
    smalls_b = (enc2_W1[:, :H], enc2_W1[:, H:],
                row(enc2_b1), row(enc2_g1), row(enc2_be1),
                enc2_W2, row(enc2_b2), row(enc2_g2), row(enc2_be2),
                enc2_W3, row(enc2_b3), row(enc2_g3), row(enc2_be3))
    in_specs = [enc_l, enc_r, enc_l, enc_r,
                att_l, att_r, att_l, att_r, att_l, att_r]
    in_specs += [full(a) for a in smalls_a]
    in_specs += [full(a) for a in smalls_b]
    in_specs += [
        full(W),
        pl.BlockSpec((2, 2, BLK, 64),
                     lambda t: (0, 0, jnp.clip(t - NBLK, 0, NBLK - 1), 0)),
        full(bc2),
    ]
    return pl.pallas_call(
        _mono_kernel,
        grid=(2 * NBLK,),
        in_specs=in_specs,
        out_specs=pl.BlockSpec((1, 2), lambda t: (0, 0)),
        out_shape=jax.ShapeDtypeStruct((1, 2), jnp.float32),
        scratch_shapes=[pltpu.VMEM((NBLK, BLK, 256), F32),
                        pltpu.VMEM((NBLK, BLK, 256), F32),
                        pltpu.VMEM((N, 64), F32),
                        pltpu.VMEM((N, 64), F32),
                        pltpu.VMEM((1, 2), F32)],
    )(x1, x1, x2, x2, adj1, adj1, adj2, adj2, alpha1, alpha1,
      *smalls_a, *smalls_b, W, Wc4, bc2)


# speedup vs baseline: 1.0528x; 1.0528x over previous
import jax
import jax.numpy as jnp
from jax.experimental import pallas as pl
from jax.experimental.pallas import tpu as pltpu

N = 2048
H = N // 2
BLK = 256
NBLK = N // BLK
F32 = jnp.float32


def _bn_relu(h, g, be):
    m = jnp.mean(h, axis=0, keepdims=True)
    v = jnp.mean((h - m) ** 2, axis=0, keepdims=True)
    return jax.nn.relu((h - m) / jnp.sqrt(v + 1e-5) * g + be)


def _enc_tail(hpre_ref, g1, be1, w2, b2, g2, be2, w3, b3, g3, be3, hs_ref):
    h = _bn_relu(hpre_ref[...].reshape(N, 256), g1[...], be1[...])
    h = jax.lax.dot_general(h, w2[...], (((1,), (1,)), ((), ())),
                            preferred_element_type=F32) + b2[...]
    h = _bn_relu(h, g2[...], be2[...])
    h = jax.lax.dot_general(h, w3[...], (((1,), (1,)), ((), ())),
                            preferred_element_type=F32) + b3[...]
    hs_ref[...] = _bn_relu(h, g3[...], be3[...])


def _mono_kernel(x1l_ref, x1r_ref, x2l_ref, x2r_ref,
                 a1l_ref, a1r_ref, a2l_ref, a2r_ref, all_ref, alr_ref,
                 w1a, b1a, g1a, be1a, w2a, b2a, g2a, be2a,
                 w3a, b3a, g3a, be3a,
                 w1b, b1b, g1b, be1b, w2b, b2b, g2b, be2b,
                 w3b, b3b, g3b, be3b,
                 w_ref, wc_ref, bc_ref,
                 out_ref, hpre1, hpre2, h1s, h2s, acc_ref):
    t = pl.program_id(0)

    @pl.when(t < NBLK)
    def _enc_step():
        dn = (((1,), (1,)), ((), ()))
        h1 = (jax.lax.dot_general(x1l_ref[...], w1a[:, :H], dn,
                                  preferred_element_type=F32) +
              jax.lax.dot_general(x1r_ref[...], w1a[:, H:], dn,
                                  preferred_element_type=F32))
        hpre1[t] = h1 + b1a[...]
        h2 = (jax.lax.dot_general(x2l_ref[...], w1b[:, :H], dn,
                                  preferred_element_type=F32) +
              jax.lax.dot_general(x2r_ref[...], w1b[:, H:], dn,
                                  preferred_element_type=F32))
        hpre2[t] = h2 + b1b[...]

    @pl.when(t == NBLK - 1)
    def _enc_tails():
        _enc_tail(hpre1, g1a, be1a, w2a, b2a, g2a, be2a,
                  w3a, b3a, g3a, be3a, h1s)
        _enc_tail(hpre2, g1b, be1b, w2b, b2b, g2b, be2b,
                  w3b, b3b, g3b, be3b, h2s)

    @pl.when(t == NBLK)
    def _init_acc():
        acc_ref[...] = jnp.zeros_like(acc_ref)

    @pl.when(t >= NBLK)
    def _attn_step():
        j = t - NBLK
        w = w_ref[...]
        dn = (((1,), (0,)), ((), ()))

        def attend(adjl, adjr, hs_ref):
            deg = (jnp.sum(adjl, axis=1, keepdims=True) +
                   jnp.sum(adjr, axis=1, keepdims=True))
            agg = (jax.lax.dot_general(all_ref[...] * adjl,
                                       hs_ref[:H, :], dn,
                                       preferred_element_type=F32) +
                   jax.lax.dot_general(alr_ref[...] * adjr,
                                       hs_ref[H:, :], dn,
                                       preferred_element_type=F32))
            inv = w / deg
            return agg * inv + hs_ref[pl.ds(j * BLK, BLK), :]

        new1 = attend(a1l_ref[...], a1r_ref[...], h1s)
        new2 = attend(a2l_ref[...], a2r_ref[...], h2s)
        contrib = (jnp.sum(wc_ref[:, 0] * new1[None], axis=(1, 2)) +
                   jnp.sum(wc_ref[:, 1] * new2[None], axis=(1, 2)))
        acc_ref[...] += contrib.reshape(1, 2)

        @pl.when(t == 2 * NBLK - 1)
        def _fin():
            logits = acc_ref[...] + bc_ref[...]
            e = jnp.exp(logits - jnp.max(logits))
            out_ref[...] = e / jnp.sum(e)


def kernel(x1, x2, adj1, adj2,
           enc1_W1, enc1_b1, enc1_g1, enc1_be1,
           enc1_W2, enc1_b2, enc1_g2, enc1_be2,
           enc1_W3, enc1_b3, enc1_g3, enc1_be3,
           enc2_W1, enc2_b1, enc2_g1, enc2_be1,
           enc2_W2, enc2_b2, enc2_g2, enc2_be2,
           enc2_W3, enc2_b3, enc2_g3, enc2_be3,
           W, alpha1, alpha2, Wc, bc):
    full = lambda a: pl.BlockSpec(a.shape, lambda t: (0,) * a.ndim)
    enc_l = pl.BlockSpec((BLK, H), lambda t: (jnp.clip(t, 0, NBLK - 1), 0))
    enc_r = pl.BlockSpec((BLK, H), lambda t: (jnp.clip(t, 0, NBLK - 1), 1))
    att_l = pl.BlockSpec((BLK, H),
                         lambda t: (jnp.clip(t - NBLK, 0, NBLK - 1), 0))
    att_r = pl.BlockSpec((BLK, H),
                         lambda t: (jnp.clip(t - NBLK, 0, NBLK - 1), 1))
    Wc4 = Wc.reshape(2, 2, N, 64)
    smalls_a = (enc1_W1, enc1_b1, enc1_g1, enc1_be1,
                enc1_W2, enc1_b2, enc1_g2, enc1_be2,
                enc1_W3, enc1_b3, enc1_g3, enc1_be3)
    smalls_b = (enc2_W1, enc2_b1, enc2_g1, enc2_be1,
                enc2_W2, enc2_b2, enc2_g2, enc2_be2,
                enc2_W3, enc2_b3, enc2_g3, enc2_be3)
    in_specs = [enc_l, enc_r, enc_l, enc_r,
                att_l, att_r, att_l, att_r, att_l, att_r]
    in_specs += [full(a) for a in smalls_a]
    in_specs += [full(a) for a in smalls_b]
    in_specs += [
        full(W),
        pl.BlockSpec((2, 2, BLK, 64),
                     lambda t: (0, 0, jnp.clip(t - NBLK, 0, NBLK - 1), 0)),
        full(bc),
    ]
    return pl.pallas_call(
        _mono_kernel,
        grid=(2 * NBLK,),
        in_specs=in_specs,
        out_specs=pl.BlockSpec((1, 2), lambda t: (0, 0)),
        out_shape=jax.ShapeDtypeStruct((1, 2), jnp.float32),
        scratch_shapes=[pltpu.VMEM((NBLK, BLK, 256), F32),
                        pltpu.VMEM((NBLK, BLK, 256), F32),
                        pltpu.VMEM((N, 64), F32),
                        pltpu.VMEM((N, 64), F32),
                        pltpu.VMEM((1, 2), F32)],
    )(x1, x1, x2, x2, adj1, adj1, adj2, adj2, alpha1, alpha1,
      *smalls_a, *smalls_b, W, Wc4, bc)
